# baseline (speedup 1.0000x reference)
import functools

import jax
import jax.numpy as jnp
import numpy as np
from jax import lax
from jax.experimental import pallas as pl
from jax.experimental.pallas import tpu as pltpu
from jax.experimental.pallas import tpu_sc as plsc

_NS = 16
_SH = 9
_ES = 16
_N = 10000
_E = 160000
_NLAYER = 4
_DIN = [16, 28, 40, 56]
_DOUT = [28, 40, 56, 56]
_DPIN = [16, 32, 48, 64]
_DPOUT = [32, 48, 64, 64]

_CH = 128
_NCHUNK = _E // _CH
_NW = 32
_CPW = 40
_NCHUNK_PAD = _NW * _CPW
_EPAD = _NCHUNK_PAD * _CH
_GRP = 4
_NGRP = _CPW // _GRP

_BE = 2048
_BN = 2000

_SC_MESH = dict(core_axis_name="c", subcore_axis_name="s",
                num_cores=2, num_subcores=16)


def _wid():
    return lax.axis_index("s") * 2 + lax.axis_index("c")


def _make_gather(widths, which):
    n_tab = len(widths)
    out_type = [jax.ShapeDtypeStruct((_EPAD, w), jnp.float32) for w in widths]
    scratch = [pltpu.VMEM((_CPW, _CH), jnp.int32),
               pltpu.VMEM((_CPW, _CH), jnp.int32)]
    for w in widths:
        scratch.append(pltpu.VMEM((_GRP * _CH, w), jnp.float32))
    scratch.append(pltpu.SemaphoreType.DMA)

    @functools.partial(
        pl.kernel,
        out_type=out_type,
        mesh=plsc.VectorSubcoreMesh(**_SC_MESH),
        scratch_types=scratch,
        compiler_params=pltpu.CompilerParams(use_tc_tiling_on_sc=False),
    )
    def gather(*refs):
        dst2d, src2d = refs[0], refs[1]
        tabs = refs[2:2 + n_tab]
        outs = refs[2 + n_tab:2 + 2 * n_tab]
        idxd = refs[2 + 2 * n_tab]
        idxs = refs[3 + 2 * n_tab]
        bufs = refs[4 + 2 * n_tab:4 + 2 * n_tab + n_tab]
        sem = refs[-1]
        wid = _wid()
        c0 = wid * _CPW

        pltpu.sync_copy(dst2d.at[pl.ds(c0, _CPW)], idxd)
        pltpu.sync_copy(src2d.at[pl.ds(c0, _CPW)], idxs)

        def group(g, carry):
            cps = []
            for k in range(_GRP):
                j = g * _GRP + k
                for t in range(n_tab):
                    idx = (idxd if which[t] == 0 else idxs).at[j]
                    cps.append(pltpu.async_copy(
                        tabs[t].at[idx],
                        bufs[t].at[pl.ds(k * _CH, _CH)], sem))
            for cp in cps:
                cp.wait()
            row0 = (c0 + g * _GRP) * _CH
            for t in range(n_tab):
                pltpu.sync_copy(bufs[t], outs[t].at[pl.ds(row0, _GRP * _CH)])
            return carry

        lax.fori_loop(0, _NGRP, group, 0)

    return gather


def _make_scatter(dp, with_counts):
    out_type = [jax.ShapeDtypeStruct((2, _N, dp), jnp.float32)]
    if with_counts:
        out_type.append(jax.ShapeDtypeStruct((2, _N, _ES), jnp.float32))
    scratch = [
        pltpu.VMEM_SHARED((_N, dp), jnp.float32),
        pltpu.VMEM((_CPW, _CH), jnp.int32),
        pltpu.VMEM((_GRP * _CH, dp), jnp.float32),
        pltpu.SemaphoreType.DMA,
    ]
    if with_counts:
        scratch.append(pltpu.VMEM_SHARED((_N, _ES), jnp.float32))
        scratch.append(pltpu.VMEM((_CH, _ES), jnp.float32))

    @functools.partial(
        pl.kernel,
        out_type=out_type,
        mesh=plsc.VectorSubcoreMesh(**_SC_MESH),
        scratch_types=scratch,
        compiler_params=pltpu.CompilerParams(use_tc_tiling_on_sc=False),
    )
    def scatter(*refs):
        if with_counts:
            (src2d, msg, zdp, z16, ones, part, cpart,
             shacc, idxst, mbuf, sem, shcnt, onesbuf) = refs
        else:
            src2d, msg, zdp, part, shacc, idxst, mbuf, sem = refs
        core = lax.axis_index("c")
        sub = lax.axis_index("s")
        wid = sub * 2 + core
        c0 = wid * _CPW

        @pl.when(sub < 10)
        def _():
            rows = pl.ds(sub * 1000, 1000)
            pltpu.sync_copy(zdp.at[rows], shacc.at[rows])
            if with_counts:
                pltpu.sync_copy(z16.at[rows], shcnt.at[rows])
        if with_counts:
            pltpu.sync_copy(ones, onesbuf)
        pltpu.sync_copy(src2d.at[pl.ds(c0, _CPW)], idxst)
        plsc.subcore_barrier()

        def group(g, carry):
            cps = []
            for k in range(_GRP):
                c = c0 + g * _GRP + k
                cps.append(pltpu.async_copy(
                    msg.at[pl.ds(c * _CH, _CH)],
                    mbuf.at[pl.ds(k * _CH, _CH)], sem))
            for cp in cps:
                cp.wait()
            for k in range(_GRP):
                c = c0 + g * _GRP + k
                j = g * _GRP + k
                pltpu.sync_copy(mbuf.at[pl.ds(k * _CH, _CH)],
                                shacc.at[idxst.at[j]], add=True)
                if with_counts:
                    @pl.when(c < _NCHUNK)
                    def _(j=j):
                        pltpu.sync_copy(onesbuf, shcnt.at[idxst.at[j]], add=True)
            return carry

        lax.fori_loop(0, _NGRP, group, 0)
        plsc.subcore_barrier()

        @pl.when(sub < 10)
        def _():
            rows = pl.ds(sub * 1000, 1000)
            pltpu.sync_copy(shacc.at[rows], part.at[core, rows])
            if with_counts:
                pltpu.sync_copy(shcnt.at[rows], cpart.at[core, rows])

    return scatter


_SH_COEF = [
    1.0,
    float(np.sqrt(3.0)), float(np.sqrt(3.0)), float(np.sqrt(3.0)),
    float(np.sqrt(15.0)), float(np.sqrt(15.0)), float(np.sqrt(5.0) / 2.0),
    float(np.sqrt(15.0)), float(np.sqrt(15.0) / 2.0),
]
_G_STOP = 5.0
_G_COEF = float(-0.5 / ((_G_STOP / (_ES - 1)) ** 2))


def _pad_row_mask(msg):
    grow = (pl.program_id(0) * _BE
            + lax.broadcasted_iota(jnp.int32, msg.shape, 0))
    return jnp.where(grow < _E, msg, 0.0)


def _edge_mlp(elen, etype, hs16, hd16, fcW1, fcb1, fcW2, fcb2):
    pre = (jnp.dot(elen, fcW1[0:16], preferred_element_type=jnp.float32)
           + jnp.dot(etype, fcW1[16:32], preferred_element_type=jnp.float32)
           + jnp.dot(hs16, fcW1[32:48], preferred_element_type=jnp.float32)
           + jnp.dot(hd16, fcW1[48:64], preferred_element_type=jnp.float32)
           + fcb1)
    hid = jnp.maximum(pre, 0.0)
    return jnp.dot(hid, fcW2, preferred_element_type=jnp.float32) + fcb2


def _tp_msg(x, sh, wtp, dpout):
    feat = jnp.concatenate([x * sh[:, k:k + 1] for k in range(_SH)], axis=1)
    return jnp.dot(feat, wtp.reshape(-1, dpout),
                   preferred_element_type=jnp.float32)


def _edge0_body(csrc_r, cdst_r, etype_r, hs_r, x_r,
                eW1_r, eb1_r, eW2_r, eb2_r,
                fcW1_r, fcb1_r, fcW2_r, fcb2_r, wtp_r,
                sh_o, elen_o, msg_o):
    ev = cdst_r[...] - csrc_r[...]
    n2 = jnp.sum(ev * ev, axis=1, keepdims=True)
    nrm = jnp.sqrt(n2)
    u = ev / jnp.maximum(nrm, 1e-8)
    ux, uy, uz = u[:, 0:1], u[:, 1:2], u[:, 2:3]
    cols = [jnp.ones_like(ux), ux, uy, uz,
            ux * uy, uy * uz, 3.0 * uz * uz - 1.0, ux * uz, ux * ux - uy * uy]
    lane = lax.broadcasted_iota(jnp.int32, (_BE, _ES), 1)
    sh = jnp.zeros((_BE, _ES), jnp.float32)
    for k in range(_SH):
        sh = jnp.where(lane == k, _SH_COEF[k] * cols[k], sh)
    sh_o[...] = sh

    offs = lane.astype(jnp.float32) * (_G_STOP / (_ES - 1))
    gg = jnp.exp(_G_COEF * (nrm - offs) ** 2)
    h1 = jnp.maximum(jnp.dot(gg, eW1_r[...],
                             preferred_element_type=jnp.float32) + eb1_r[...], 0.0)
    elen = jnp.dot(h1, eW2_r[...], preferred_element_type=jnp.float32) + eb2_r[...]
    elen_o[...] = elen

    x = x_r[...]
    w = _edge_mlp(elen, etype_r[...], hs_r[...], x[:, :16],
                  fcW1_r[...], fcb1_r[...], fcW2_r[...], fcb2_r[...])
    msg_o[...] = _pad_row_mask(_tp_msg(x, sh, wtp_r[...], _DPOUT[0]) * w)


def _edgeL_body(l, sh_r, elen_r, etype_r, hs_r, x_r,
                fcW1_r, fcb1_r, fcW2_r, fcb2_r, wtp_r, msg_o):
    x = x_r[...]
    sh = sh_r[...]
    w = _edge_mlp(elen_r[...], etype_r[...], hs_r[...], x[:, :16],
                  fcW1_r[...], fcb1_r[...], fcW2_r[...], fcb2_r[...])
    msg_o[...] = _pad_row_mask(_tp_msg(x, sh, wtp_r[...], _DPOUT[l]) * w)


def _full_spec(shape):
    return pl.BlockSpec(shape, lambda i: (0,) * len(shape))


def _rows_spec(w):
    return pl.BlockSpec((_BE, w), lambda i: (i, 0))


def _make_edge0(dpout):
    grid = (_EPAD // _BE,)
    in_specs = [_rows_spec(_ES)] * 5 + [
        _full_spec((_ES, _ES)), _full_spec((1, _ES)),
        _full_spec((_ES, _ES)), _full_spec((1, _ES)),
        _full_spec((64, 64)), _full_spec((1, 64)),
        _full_spec((64, dpout)), _full_spec((1, dpout)),
        _full_spec((_SH, _DPIN[0], dpout)),
    ]
    out_specs = [_rows_spec(_ES), _rows_spec(_ES), _rows_spec(dpout)]
    out_shape = [jax.ShapeDtypeStruct((_EPAD, _ES), jnp.float32),
                 jax.ShapeDtypeStruct((_EPAD, _ES), jnp.float32),
                 jax.ShapeDtypeStruct((_EPAD, dpout), jnp.float32)]
    return pl.pallas_call(_edge0_body, grid=grid, in_specs=in_specs,
                          out_specs=out_specs, out_shape=out_shape)


def _make_edgeL(l):
    dpin, dpout = _DPIN[l], _DPOUT[l]
    grid = (_EPAD // _BE,)
    in_specs = [_rows_spec(_ES)] * 4 + [
        _rows_spec(dpin),
        _full_spec((64, 64)), _full_spec((1, 64)),
        _full_spec((64, dpout)), _full_spec((1, dpout)),
        _full_spec((_SH, dpin, dpout)),
    ]
    out_specs = [_rows_spec(dpout)]
    out_shape = [jax.ShapeDtypeStruct((_EPAD, dpout), jnp.float32)]
    return pl.pallas_call(functools.partial(_edgeL_body, l), grid=grid,
                          in_specs=in_specs, out_specs=out_specs,
                          out_shape=out_shape)


def _norm_update(part, cnt, gamma, beta, dout, dpout):
    agg = part[0] + part[1]
    out = agg / cnt
    mu = jnp.sum(out, axis=1, keepdims=True) * (1.0 / dout)
    lane = lax.broadcasted_iota(jnp.int32, out.shape, 1)
    dev = jnp.where(lane < dout, out - mu, 0.0)
    var = jnp.sum(dev * dev, axis=1, keepdims=True) * (1.0 / dout)
    return dev * lax.rsqrt(var + 1e-5) * gamma + beta


def _node0_body(part_r, cpart_r, h_r, gamma_r, beta_r,
                h_o, h16_o, cnt_o):
    cnt = jnp.maximum(cpart_r[0] + cpart_r[1], 1.0)
    cnt_o[...] = cnt
    upd = _norm_update(part_r[...], cnt[:, 0:1], gamma_r[...], beta_r[...],
                       _DOUT[0], _DPOUT[0])
    hold = h_r[...]
    pad = _DPOUT[0] - hold.shape[1]
    hnew = jnp.concatenate(
        [hold, jnp.zeros((hold.shape[0], pad), jnp.float32)], axis=1) + upd
    h_o[...] = hnew
    h16_o[...] = hnew[:, :16]


def _nodeL_body(l, part_r, cnt_r, h_r, gamma_r, beta_r, h_o, h16_o):
    upd = _norm_update(part_r[...], cnt_r[:, 0:1], gamma_r[...], beta_r[...],
                       _DOUT[l], _DPOUT[l])
    hold = h_r[...]
    pad = _DPOUT[l] - hold.shape[1]
    if pad:
        hold = jnp.concatenate(
            [hold, jnp.zeros((hold.shape[0], pad), jnp.float32)], axis=1)
    hnew = hold + upd
    h_o[...] = hnew
    h16_o[...] = hnew[:, :16]


def _node3_body(part_r, cnt_r, h_r, gamma_r, beta_r,
                oW1_r, ob1_r, oW2_r, ob2_r, out_o):
    upd = _norm_update(part_r[...], cnt_r[:, 0:1], gamma_r[...], beta_r[...],
                       _DOUT[3], _DPOUT[3])
    hnew = h_r[...] + upd
    emb = jnp.concatenate([hnew[:, :16], hnew[:, 40:56]], axis=1)
    e1 = jnp.maximum(jnp.dot(emb, oW1_r[...],
                             preferred_element_type=jnp.float32) + ob1_r[...], 0.0)
    out_o[...] = jnp.dot(e1, oW2_r[...],
                         preferred_element_type=jnp.float32) + ob2_r[...]


def _nrows_spec(w):
    return pl.BlockSpec((_BN, w), lambda i: (i, 0))


def _part_spec(w):
    return pl.BlockSpec((2, _BN, w), lambda i: (0, i, 0))


def _make_node0():
    dpo = _DPOUT[0]
    grid = (_N // _BN,)
    in_specs = [_part_spec(dpo), _part_spec(_ES), _nrows_spec(_DPIN[0]),
                _full_spec((1, dpo)), _full_spec((1, dpo))]
    out_specs = [_nrows_spec(dpo), _nrows_spec(_ES), _nrows_spec(_ES)]
    out_shape = [jax.ShapeDtypeStruct((_N, dpo), jnp.float32),
                 jax.ShapeDtypeStruct((_N, _ES), jnp.float32),
                 jax.ShapeDtypeStruct((_N, _ES), jnp.float32)]
    return pl.pallas_call(_node0_body, grid=grid, in_specs=in_specs,
                          out_specs=out_specs, out_shape=out_shape)


def _make_nodeL(l):
    dpo = _DPOUT[l]
    grid = (_N // _BN,)
    in_specs = [_part_spec(dpo), _nrows_spec(_ES), _nrows_spec(_DPIN[l]),
                _full_spec((1, dpo)), _full_spec((1, dpo))]
    out_specs = [_nrows_spec(dpo), _nrows_spec(_ES)]
    out_shape = [jax.ShapeDtypeStruct((_N, dpo), jnp.float32),
                 jax.ShapeDtypeStruct((_N, _ES), jnp.float32)]
    return pl.pallas_call(functools.partial(_nodeL_body, l), grid=grid,
                          in_specs=in_specs, out_specs=out_specs,
                          out_shape=out_shape)


def _make_node3():
    dpo = _DPOUT[3]
    grid = (_N // _BN,)
    in_specs = [_part_spec(dpo), _nrows_spec(_ES), _nrows_spec(_DPIN[3]),
                _full_spec((1, dpo)), _full_spec((1, dpo)),
                _full_spec((2 * _NS, 2 * _NS)), _full_spec((1, 2 * _NS)),
                _full_spec((2 * _NS, _NS)), _full_spec((1, _NS))]
    out_specs = [_nrows_spec(_NS)]
    out_shape = [jax.ShapeDtypeStruct((_N, _NS), jnp.float32)]
    return pl.pallas_call(_node3_body, grid=grid, in_specs=in_specs,
                          out_specs=out_specs, out_shape=out_shape)


def _pad_cols(a, w):
    return jnp.pad(a, ((0, 0), (0, w - a.shape[1])))


def kernel(node_attr, coords, batch_id, perturb_mask, edges, edge_type_attr, params):
    src = edges[0].astype(jnp.int32)
    dst = edges[1].astype(jnp.int32)
    zpad = jnp.zeros((_EPAD - _E,), jnp.int32)
    src2d = jnp.concatenate([src, zpad]).reshape(_NCHUNK_PAD, _CH)
    dst2d = jnp.concatenate([dst, zpad]).reshape(_NCHUNK_PAD, _CH)
    coords_p = _pad_cols(coords.astype(jnp.float32), _ES)
    nat = node_attr.astype(jnp.float32)
    etype_p = jnp.pad(edge_type_attr, ((0, _EPAD - _E), (0, 0)))

    lw = []
    for l in range(_NLAYER):
        lp = params['layers']['l%d' % l]
        din, dout = _DIN[l], _DOUT[l]
        dpin, dpo = _DPIN[l], _DPOUT[l]
        wtp = lp['Wtp'].reshape(din, _SH, dout).transpose(1, 0, 2)
        wtp = jnp.pad(wtp, ((0, 0), (0, dpin - din), (0, dpo - dout)))
        lw.append(dict(
            fcW1=lp['fcW1'],
            fcb1=lp['fcb1'][None, :],
            fcW2=_pad_cols(lp['fcW2'], dpo),
            fcb2=_pad_cols(lp['fcb2'][None, :], dpo),
            wtp=wtp,
            gamma=_pad_cols(lp['gamma'][None, :], dpo),
            beta=_pad_cols(lp['beta'][None, :], dpo),
        ))
    ee = params['edge_emb']
    oo = params['out_ffn']

    ones16 = jnp.ones((_CH, _ES), jnp.float32)
    z16 = jnp.zeros((_N, _ES), jnp.float32)

    cdst, csrc, x0, hs0 = _make_gather([_ES] * 4, [0, 1, 0, 1])(
        dst2d, src2d, coords_p, coords_p, nat, nat)
    sh, elen, msg0 = _make_edge0(_DPOUT[0])(
        csrc, cdst, etype_p, hs0, x0,
        ee['W1'], ee['b1'][None, :], ee['W2'], ee['b2'][None, :],
        lw[0]['fcW1'], lw[0]['fcb1'], lw[0]['fcW2'], lw[0]['fcb2'],
        lw[0]['wtp'])
    part0, cpart = _make_scatter(_DPOUT[0], True)(
        src2d, msg0, jnp.zeros((_N, _DPOUT[0]), jnp.float32), z16, ones16)
    h, h16, cnt = _make_node0()(part0, cpart, nat, lw[0]['gamma'], lw[0]['beta'])

    for l in (1, 2):
        x, hs = _make_gather([_DPIN[l], _ES], [0, 1])(dst2d, src2d, h, h16)
        (msg,) = _make_edgeL(l)(
            sh, elen, etype_p, hs, x,
            lw[l]['fcW1'], lw[l]['fcb1'], lw[l]['fcW2'], lw[l]['fcb2'],
            lw[l]['wtp'])
        (part,) = _make_scatter(_DPOUT[l], False)(
            src2d, msg, jnp.zeros((_N, _DPOUT[l]), jnp.float32))
        h, h16 = _make_nodeL(l)(part, cnt, h, lw[l]['gamma'], lw[l]['beta'])

    x, hs = _make_gather([_DPIN[3], _ES], [0, 1])(dst2d, src2d, h, h16)
    (msg,) = _make_edgeL(3)(
        sh, elen, etype_p, hs, x,
        lw[3]['fcW1'], lw[3]['fcb1'], lw[3]['fcW2'], lw[3]['fcb2'],
        lw[3]['wtp'])
    (part,) = _make_scatter(_DPOUT[3], False)(
        src2d, msg, jnp.zeros((_N, _DPOUT[3]), jnp.float32))
    (out,) = _make_node3()(part, cnt, h, lw[3]['gamma'], lw[3]['beta'],
                           oo['W1'], oo['b1'][None, :], oo['W2'], oo['b2'][None, :])
    return out


# speedup vs baseline: 1.1516x; 1.1516x over previous
import functools

import jax
import jax.numpy as jnp
import numpy as np
from jax import lax
from jax.experimental import pallas as pl
from jax.experimental.pallas import tpu as pltpu
from jax.experimental.pallas import tpu_sc as plsc

_NS = 16
_SH = 9
_ES = 16
_N = 10000
_E = 160000
_NLAYER = 4
_DIN = [16, 28, 40, 56]
_DOUT = [28, 40, 56, 56]
_DPIN = [16, 32, 48, 64]
_DPOUT = [32, 48, 64, 64]

_CH = 128
_NCHUNK = _E // _CH
_NW = 32
_CPW = 40
_NCHUNK_PAD = _NW * _CPW
_EPAD = _NCHUNK_PAD * _CH
_GRP = 4
_NGRP = _CPW // _GRP

_BE = 2048
_BN = 2000

_SC_MESH = dict(core_axis_name="c", subcore_axis_name="s",
                num_cores=2, num_subcores=16)


def _wid():
    return lax.axis_index("s") * 2 + lax.axis_index("c")


def _make_gather(widths, which):
    n_tab = len(widths)
    out_type = [jax.ShapeDtypeStruct((_EPAD, w), jnp.float32) for w in widths]
    scratch = [pltpu.VMEM((_CPW, _CH), jnp.int32),
               pltpu.VMEM((_CPW, _CH), jnp.int32)]
    for w in widths:
        scratch.append(pltpu.VMEM((_GRP * _CH, w), jnp.float32))
    scratch.append(pltpu.SemaphoreType.DMA)

    @functools.partial(
        pl.kernel,
        out_type=out_type,
        mesh=plsc.VectorSubcoreMesh(**_SC_MESH),
        scratch_types=scratch,
        compiler_params=pltpu.CompilerParams(use_tc_tiling_on_sc=False),
    )
    def gather(*refs):
        dst2d, src2d = refs[0], refs[1]
        tabs = refs[2:2 + n_tab]
        outs = refs[2 + n_tab:2 + 2 * n_tab]
        idxd = refs[2 + 2 * n_tab]
        idxs = refs[3 + 2 * n_tab]
        bufs = refs[4 + 2 * n_tab:4 + 2 * n_tab + n_tab]
        sem = refs[-1]
        wid = _wid()
        c0 = wid * _CPW

        pltpu.sync_copy(dst2d.at[pl.ds(c0, _CPW)], idxd)
        pltpu.sync_copy(src2d.at[pl.ds(c0, _CPW)], idxs)

        def group(g, carry):
            cps = []
            for k in range(_GRP):
                j = g * _GRP + k
                for t in range(n_tab):
                    idx = (idxd if which[t] == 0 else idxs).at[j]
                    cps.append(pltpu.async_copy(
                        tabs[t].at[idx],
                        bufs[t].at[pl.ds(k * _CH, _CH)], sem))
            for cp in cps:
                cp.wait()
            row0 = (c0 + g * _GRP) * _CH
            for t in range(n_tab):
                pltpu.sync_copy(bufs[t], outs[t].at[pl.ds(row0, _GRP * _CH)])
            return carry

        lax.fori_loop(0, _NGRP, group, 0)

    return gather


def _make_scatter(dp, with_counts):
    out_type = [jax.ShapeDtypeStruct((2, _N, dp), jnp.float32)]
    if with_counts:
        out_type.append(jax.ShapeDtypeStruct((2, _N, _ES), jnp.float32))
    scratch = [
        pltpu.VMEM_SHARED((_N, dp), jnp.float32),
        pltpu.VMEM((_CPW, _CH), jnp.int32),
        pltpu.VMEM((_GRP * _CH, dp), jnp.float32),
        pltpu.SemaphoreType.DMA,
    ]
    if with_counts:
        scratch.append(pltpu.VMEM_SHARED((_N, _ES), jnp.float32))
        scratch.append(pltpu.VMEM((_CH, _ES), jnp.float32))

    @functools.partial(
        pl.kernel,
        out_type=out_type,
        mesh=plsc.VectorSubcoreMesh(**_SC_MESH),
        scratch_types=scratch,
        compiler_params=pltpu.CompilerParams(use_tc_tiling_on_sc=False),
    )
    def scatter(*refs):
        if with_counts:
            (src2d, msg, zdp, z16, ones, part, cpart,
             shacc, idxst, mbuf, sem, shcnt, onesbuf) = refs
        else:
            src2d, msg, zdp, part, shacc, idxst, mbuf, sem = refs
        core = lax.axis_index("c")
        sub = lax.axis_index("s")
        wid = sub * 2 + core
        c0 = wid * _CPW

        @pl.when(sub < 10)
        def _():
            rows = pl.ds(sub * 1000, 1000)
            pltpu.sync_copy(zdp.at[rows], shacc.at[rows])
            if with_counts:
                pltpu.sync_copy(z16.at[rows], shcnt.at[rows])
        if with_counts:
            pltpu.sync_copy(ones, onesbuf)
        pltpu.sync_copy(src2d.at[pl.ds(c0, _CPW)], idxst)
        plsc.subcore_barrier()

        def group(g, carry):
            cps = []
            for k in range(_GRP):
                c = c0 + g * _GRP + k
                cps.append(pltpu.async_copy(
                    msg.at[pl.ds(c * _CH, _CH)],
                    mbuf.at[pl.ds(k * _CH, _CH)], sem))
            for cp in cps:
                cp.wait()
            for k in range(_GRP):
                c = c0 + g * _GRP + k
                j = g * _GRP + k
                pltpu.sync_copy(mbuf.at[pl.ds(k * _CH, _CH)],
                                shacc.at[idxst.at[j]], add=True)
                if with_counts:
                    @pl.when(c < _NCHUNK)
                    def _(j=j):
                        pltpu.sync_copy(onesbuf, shcnt.at[idxst.at[j]], add=True)
            return carry

        lax.fori_loop(0, _NGRP, group, 0)
        plsc.subcore_barrier()

        @pl.when(sub < 10)
        def _():
            rows = pl.ds(sub * 1000, 1000)
            pltpu.sync_copy(shacc.at[rows], part.at[core, rows])
            if with_counts:
                pltpu.sync_copy(shcnt.at[rows], cpart.at[core, rows])

    return scatter


_SH_COEF = [
    1.0,
    float(np.sqrt(3.0)), float(np.sqrt(3.0)), float(np.sqrt(3.0)),
    float(np.sqrt(15.0)), float(np.sqrt(15.0)), float(np.sqrt(5.0) / 2.0),
    float(np.sqrt(15.0)), float(np.sqrt(15.0) / 2.0),
]
_G_STOP = 5.0
_G_COEF = float(-0.5 / ((_G_STOP / (_ES - 1)) ** 2))


def _pad_row_mask(msg):
    grow = (pl.program_id(0) * _BE
            + lax.broadcasted_iota(jnp.int32, msg.shape, 0))
    return jnp.where(grow < _E, msg, 0.0)


def _edge_mlp(elen, etype, hs16, hd16, fcW1, fcb1, fcW2, fcb2):
    pre = (jnp.dot(elen, fcW1[0:16], preferred_element_type=jnp.float32)
           + jnp.dot(etype, fcW1[16:32], preferred_element_type=jnp.float32)
           + jnp.dot(hs16, fcW1[32:48], preferred_element_type=jnp.float32)
           + jnp.dot(hd16, fcW1[48:64], preferred_element_type=jnp.float32)
           + fcb1)
    hid = jnp.maximum(pre, 0.0)
    return jnp.dot(hid, fcW2, preferred_element_type=jnp.float32) + fcb2


def _tp_msg(x, sh, wtp, dpout):
    acc = jnp.dot(x * sh[:, 0:1], wtp[0], preferred_element_type=jnp.float32)
    for k in range(1, _SH):
        acc = acc + jnp.dot(x * sh[:, k:k + 1], wtp[k],
                            preferred_element_type=jnp.float32)
    return acc


def _edge0_body(csrc_r, cdst_r, etype_r, hs_r, x_r,
                eW1_r, eb1_r, eW2_r, eb2_r,
                fcW1_r, fcb1_r, fcW2_r, fcb2_r, wtp_r,
                sh_o, elen_o, msg_o):
    ev = cdst_r[...] - csrc_r[...]
    n2 = jnp.sum(ev * ev, axis=1, keepdims=True)
    nrm = jnp.sqrt(n2)
    u = ev / jnp.maximum(nrm, 1e-8)
    ux, uy, uz = u[:, 0:1], u[:, 1:2], u[:, 2:3]
    cols = [jnp.ones_like(ux), ux, uy, uz,
            ux * uy, uy * uz, 3.0 * uz * uz - 1.0, ux * uz, ux * ux - uy * uy]
    lane = lax.broadcasted_iota(jnp.int32, (_BE, _ES), 1)
    sh = jnp.zeros((_BE, _ES), jnp.float32)
    for k in range(_SH):
        sh = jnp.where(lane == k, _SH_COEF[k] * cols[k], sh)
    sh_o[...] = sh

    offs = lane.astype(jnp.float32) * (_G_STOP / (_ES - 1))
    gg = jnp.exp(_G_COEF * (nrm - offs) ** 2)
    h1 = jnp.maximum(jnp.dot(gg, eW1_r[...],
                             preferred_element_type=jnp.float32) + eb1_r[...], 0.0)
    elen = jnp.dot(h1, eW2_r[...], preferred_element_type=jnp.float32) + eb2_r[...]
    elen_o[...] = elen

    x = x_r[...]
    w = _edge_mlp(elen, etype_r[...], hs_r[...], x[:, :16],
                  fcW1_r[...], fcb1_r[...], fcW2_r[...], fcb2_r[...])
    msg_o[...] = _pad_row_mask(_tp_msg(x, sh, wtp_r[...], _DPOUT[0]) * w)


def _edgeL_body(l, sh_r, elen_r, etype_r, hs_r, x_r,
                fcW1_r, fcb1_r, fcW2_r, fcb2_r, wtp_r, msg_o):
    x = x_r[...]
    sh = sh_r[...]
    w = _edge_mlp(elen_r[...], etype_r[...], hs_r[...], x[:, :16],
                  fcW1_r[...], fcb1_r[...], fcW2_r[...], fcb2_r[...])
    msg_o[...] = _pad_row_mask(_tp_msg(x, sh, wtp_r[...], _DPOUT[l]) * w)


def _full_spec(shape):
    return pl.BlockSpec(shape, lambda i: (0,) * len(shape))


def _rows_spec(w):
    return pl.BlockSpec((_BE, w), lambda i: (i, 0))


def _make_edge0(dpout):
    grid = (_EPAD // _BE,)
    in_specs = [_rows_spec(_ES)] * 5 + [
        _full_spec((_ES, _ES)), _full_spec((1, _ES)),
        _full_spec((_ES, _ES)), _full_spec((1, _ES)),
        _full_spec((64, 64)), _full_spec((1, 64)),
        _full_spec((64, dpout)), _full_spec((1, dpout)),
        _full_spec((_SH, _DPIN[0], dpout)),
    ]
    out_specs = [_rows_spec(_ES), _rows_spec(_ES), _rows_spec(dpout)]
    out_shape = [jax.ShapeDtypeStruct((_EPAD, _ES), jnp.float32),
                 jax.ShapeDtypeStruct((_EPAD, _ES), jnp.float32),
                 jax.ShapeDtypeStruct((_EPAD, dpout), jnp.float32)]
    return pl.pallas_call(_edge0_body, grid=grid, in_specs=in_specs,
                          out_specs=out_specs, out_shape=out_shape)


def _make_edgeL(l):
    dpin, dpout = _DPIN[l], _DPOUT[l]
    grid = (_EPAD // _BE,)
    in_specs = [_rows_spec(_ES)] * 4 + [
        _rows_spec(dpin),
        _full_spec((64, 64)), _full_spec((1, 64)),
        _full_spec((64, dpout)), _full_spec((1, dpout)),
        _full_spec((_SH, dpin, dpout)),
    ]
    out_specs = [_rows_spec(dpout)]
    out_shape = [jax.ShapeDtypeStruct((_EPAD, dpout), jnp.float32)]
    return pl.pallas_call(functools.partial(_edgeL_body, l), grid=grid,
                          in_specs=in_specs, out_specs=out_specs,
                          out_shape=out_shape)


def _norm_update(part, cnt, gamma, beta, dout, dpout):
    agg = part[0] + part[1]
    out = agg / cnt
    mu = jnp.sum(out, axis=1, keepdims=True) * (1.0 / dout)
    lane = lax.broadcasted_iota(jnp.int32, out.shape, 1)
    dev = jnp.where(lane < dout, out - mu, 0.0)
    var = jnp.sum(dev * dev, axis=1, keepdims=True) * (1.0 / dout)
    return dev * lax.rsqrt(var + 1e-5) * gamma + beta


def _node0_body(part_r, cpart_r, h_r, gamma_r, beta_r,
                h_o, h16_o, cnt_o):
    cnt = jnp.maximum(cpart_r[0] + cpart_r[1], 1.0)
    cnt_o[...] = cnt
    upd = _norm_update(part_r[...], cnt[:, 0:1], gamma_r[...], beta_r[...],
                       _DOUT[0], _DPOUT[0])
    hold = h_r[...]
    pad = _DPOUT[0] - hold.shape[1]
    hnew = jnp.concatenate(
        [hold, jnp.zeros((hold.shape[0], pad), jnp.float32)], axis=1) + upd
    h_o[...] = hnew
    h16_o[...] = hnew[:, :16]


def _nodeL_body(l, part_r, cnt_r, h_r, gamma_r, beta_r, h_o, h16_o):
    upd = _norm_update(part_r[...], cnt_r[:, 0:1], gamma_r[...], beta_r[...],
                       _DOUT[l], _DPOUT[l])
    hold = h_r[...]
    pad = _DPOUT[l] - hold.shape[1]
    if pad:
        hold = jnp.concatenate(
            [hold, jnp.zeros((hold.shape[0], pad), jnp.float32)], axis=1)
    hnew = hold + upd
    h_o[...] = hnew
    h16_o[...] = hnew[:, :16]


def _node3_body(part_r, cnt_r, h_r, gamma_r, beta_r,
                oW1_r, ob1_r, oW2_r, ob2_r, out_o):
    upd = _norm_update(part_r[...], cnt_r[:, 0:1], gamma_r[...], beta_r[...],
                       _DOUT[3], _DPOUT[3])
    hnew = h_r[...] + upd
    emb = jnp.concatenate([hnew[:, :16], hnew[:, 40:56]], axis=1)
    e1 = jnp.maximum(jnp.dot(emb, oW1_r[...],
                             preferred_element_type=jnp.float32) + ob1_r[...], 0.0)
    out_o[...] = jnp.dot(e1, oW2_r[...],
                         preferred_element_type=jnp.float32) + ob2_r[...]


def _nrows_spec(w):
    return pl.BlockSpec((_BN, w), lambda i: (i, 0))


def _part_spec(w):
    return pl.BlockSpec((2, _BN, w), lambda i: (0, i, 0))


def _make_node0():
    dpo = _DPOUT[0]
    grid = (_N // _BN,)
    in_specs = [_part_spec(dpo), _part_spec(_ES), _nrows_spec(_DPIN[0]),
                _full_spec((1, dpo)), _full_spec((1, dpo))]
    out_specs = [_nrows_spec(dpo), _nrows_spec(_ES), _nrows_spec(_ES)]
    out_shape = [jax.ShapeDtypeStruct((_N, dpo), jnp.float32),
                 jax.ShapeDtypeStruct((_N, _ES), jnp.float32),
                 jax.ShapeDtypeStruct((_N, _ES), jnp.float32)]
    return pl.pallas_call(_node0_body, grid=grid, in_specs=in_specs,
                          out_specs=out_specs, out_shape=out_shape)


def _make_nodeL(l):
    dpo = _DPOUT[l]
    grid = (_N // _BN,)
    in_specs = [_part_spec(dpo), _nrows_spec(_ES), _nrows_spec(_DPIN[l]),
                _full_spec((1, dpo)), _full_spec((1, dpo))]
    out_specs = [_nrows_spec(dpo), _nrows_spec(_ES)]
    out_shape = [jax.ShapeDtypeStruct((_N, dpo), jnp.float32),
                 jax.ShapeDtypeStruct((_N, _ES), jnp.float32)]
    return pl.pallas_call(functools.partial(_nodeL_body, l), grid=grid,
                          in_specs=in_specs, out_specs=out_specs,
                          out_shape=out_shape)


def _make_node3():
    dpo = _DPOUT[3]
    grid = (_N // _BN,)
    in_specs = [_part_spec(dpo), _nrows_spec(_ES), _nrows_spec(_DPIN[3]),
                _full_spec((1, dpo)), _full_spec((1, dpo)),
                _full_spec((2 * _NS, 2 * _NS)), _full_spec((1, 2 * _NS)),
                _full_spec((2 * _NS, _NS)), _full_spec((1, _NS))]
    out_specs = [_nrows_spec(_NS)]
    out_shape = [jax.ShapeDtypeStruct((_N, _NS), jnp.float32)]
    return pl.pallas_call(_node3_body, grid=grid, in_specs=in_specs,
                          out_specs=out_specs, out_shape=out_shape)


def _pad_cols(a, w):
    return jnp.pad(a, ((0, 0), (0, w - a.shape[1])))


def kernel(node_attr, coords, batch_id, perturb_mask, edges, edge_type_attr, params):
    src = edges[0].astype(jnp.int32)
    dst = edges[1].astype(jnp.int32)
    zpad = jnp.zeros((_EPAD - _E,), jnp.int32)
    src2d = jnp.concatenate([src, zpad]).reshape(_NCHUNK_PAD, _CH)
    dst2d = jnp.concatenate([dst, zpad]).reshape(_NCHUNK_PAD, _CH)
    coords_p = _pad_cols(coords.astype(jnp.float32), _ES)
    nat = node_attr.astype(jnp.float32)
    etype_p = jnp.pad(edge_type_attr, ((0, _EPAD - _E), (0, 0)))

    lw = []
    for l in range(_NLAYER):
        lp = params['layers']['l%d' % l]
        din, dout = _DIN[l], _DOUT[l]
        dpin, dpo = _DPIN[l], _DPOUT[l]
        wtp = lp['Wtp'].reshape(din, _SH, dout).transpose(1, 0, 2)
        wtp = jnp.pad(wtp, ((0, 0), (0, dpin - din), (0, dpo - dout)))
        lw.append(dict(
            fcW1=lp['fcW1'],
            fcb1=lp['fcb1'][None, :],
            fcW2=_pad_cols(lp['fcW2'], dpo),
            fcb2=_pad_cols(lp['fcb2'][None, :], dpo),
            wtp=wtp,
            gamma=_pad_cols(lp['gamma'][None, :], dpo),
            beta=_pad_cols(lp['beta'][None, :], dpo),
        ))
    ee = params['edge_emb']
    oo = params['out_ffn']

    ones16 = jnp.ones((_CH, _ES), jnp.float32)
    z16 = jnp.zeros((_N, _ES), jnp.float32)

    cdst, csrc, x0, hs0 = _make_gather([_ES] * 4, [0, 1, 0, 1])(
        dst2d, src2d, coords_p, coords_p, nat, nat)
    sh, elen, msg0 = _make_edge0(_DPOUT[0])(
        csrc, cdst, etype_p, hs0, x0,
        ee['W1'], ee['b1'][None, :], ee['W2'], ee['b2'][None, :],
        lw[0]['fcW1'], lw[0]['fcb1'], lw[0]['fcW2'], lw[0]['fcb2'],
        lw[0]['wtp'])
    part0, cpart = _make_scatter(_DPOUT[0], True)(
        src2d, msg0, jnp.zeros((_N, _DPOUT[0]), jnp.float32), z16, ones16)
    h, h16, cnt = _make_node0()(part0, cpart, nat, lw[0]['gamma'], lw[0]['beta'])

    for l in (1, 2):
        x, hs = _make_gather([_DPIN[l], _ES], [0, 1])(dst2d, src2d, h, h16)
        (msg,) = _make_edgeL(l)(
            sh, elen, etype_p, hs, x,
            lw[l]['fcW1'], lw[l]['fcb1'], lw[l]['fcW2'], lw[l]['fcb2'],
            lw[l]['wtp'])
        (part,) = _make_scatter(_DPOUT[l], False)(
            src2d, msg, jnp.zeros((_N, _DPOUT[l]), jnp.float32))
        h, h16 = _make_nodeL(l)(part, cnt, h, lw[l]['gamma'], lw[l]['beta'])

    x, hs = _make_gather([_DPIN[3], _ES], [0, 1])(dst2d, src2d, h, h16)
    (msg,) = _make_edgeL(3)(
        sh, elen, etype_p, hs, x,
        lw[3]['fcW1'], lw[3]['fcb1'], lw[3]['fcW2'], lw[3]['fcb2'],
        lw[3]['wtp'])
    (part,) = _make_scatter(_DPOUT[3], False)(
        src2d, msg, jnp.zeros((_N, _DPOUT[3]), jnp.float32))
    (out,) = _make_node3()(part, cnt, h, lw[3]['gamma'], lw[3]['beta'],
                           oo['W1'], oo['b1'][None, :], oo['W2'], oo['b2'][None, :])
    return out


# speedup vs baseline: 1.1782x; 1.0230x over previous
import functools

import jax
import jax.numpy as jnp
import numpy as np
from jax import lax
from jax.experimental import pallas as pl
from jax.experimental.pallas import tpu as pltpu
from jax.experimental.pallas import tpu_sc as plsc

_NS = 16
_SH = 9
_ES = 16
_N = 10000
_E = 160000
_NLAYER = 4
_DIN = [16, 28, 40, 56]
_DOUT = [28, 40, 56, 56]
_DPIN = [16, 32, 48, 64]
_DPOUT = [32, 48, 64, 64]

_CH = 128
_NCHUNK = _E // _CH
_NW = 32
_CPW = 40
_NCHUNK_PAD = _NW * _CPW
_EPAD = _NCHUNK_PAD * _CH
_GRP = 8
_NGRP = _CPW // _GRP

_BE = 4096
_BN = 2000

_SC_MESH = dict(core_axis_name="c", subcore_axis_name="s",
                num_cores=2, num_subcores=16)


def _wid():
    return lax.axis_index("s") * 2 + lax.axis_index("c")


def _make_gather(widths, which):
    n_tab = len(widths)
    out_type = [jax.ShapeDtypeStruct((_EPAD, w), jnp.float32) for w in widths]
    scratch = [pltpu.VMEM((_CPW, _CH), jnp.int32),
               pltpu.VMEM((_CPW, _CH), jnp.int32)]
    for w in widths:
        scratch.append(pltpu.VMEM((_GRP * _CH, w), jnp.float32))
    scratch.append(pltpu.SemaphoreType.DMA)

    @functools.partial(
        pl.kernel,
        out_type=out_type,
        mesh=plsc.VectorSubcoreMesh(**_SC_MESH),
        scratch_types=scratch,
        compiler_params=pltpu.CompilerParams(use_tc_tiling_on_sc=False),
    )
    def gather(*refs):
        dst2d, src2d = refs[0], refs[1]
        tabs = refs[2:2 + n_tab]
        outs = refs[2 + n_tab:2 + 2 * n_tab]
        idxd = refs[2 + 2 * n_tab]
        idxs = refs[3 + 2 * n_tab]
        bufs = refs[4 + 2 * n_tab:4 + 2 * n_tab + n_tab]
        sem = refs[-1]
        wid = _wid()
        c0 = wid * _CPW

        pltpu.sync_copy(dst2d.at[pl.ds(c0, _CPW)], idxd)
        pltpu.sync_copy(src2d.at[pl.ds(c0, _CPW)], idxs)

        def group(g, carry):
            cps = []
            for k in range(_GRP):
                j = g * _GRP + k
                for t in range(n_tab):
                    idx = (idxd if which[t] == 0 else idxs).at[j]
                    cps.append(pltpu.async_copy(
                        tabs[t].at[idx],
                        bufs[t].at[pl.ds(k * _CH, _CH)], sem))
            for cp in cps:
                cp.wait()
            row0 = (c0 + g * _GRP) * _CH
            for t in range(n_tab):
                pltpu.sync_copy(bufs[t], outs[t].at[pl.ds(row0, _GRP * _CH)])
            return carry

        lax.fori_loop(0, _NGRP, group, 0)

    return gather


def _make_scatter(dp, with_counts):
    out_type = [jax.ShapeDtypeStruct((2, _N, dp), jnp.float32)]
    if with_counts:
        out_type.append(jax.ShapeDtypeStruct((2, _N, _ES), jnp.float32))
    scratch = [
        pltpu.VMEM_SHARED((_N, dp), jnp.float32),
        pltpu.VMEM((_CPW, _CH), jnp.int32),
        pltpu.VMEM((_GRP * _CH, dp), jnp.float32),
        pltpu.SemaphoreType.DMA,
    ]
    if with_counts:
        scratch.append(pltpu.VMEM_SHARED((_N, _ES), jnp.float32))
        scratch.append(pltpu.VMEM((_CH, _ES), jnp.float32))

    @functools.partial(
        pl.kernel,
        out_type=out_type,
        mesh=plsc.VectorSubcoreMesh(**_SC_MESH),
        scratch_types=scratch,
        compiler_params=pltpu.CompilerParams(use_tc_tiling_on_sc=False),
    )
    def scatter(*refs):
        if with_counts:
            (src2d, msg, zdp, z16, ones, part, cpart,
             shacc, idxst, mbuf, sem, shcnt, onesbuf) = refs
        else:
            src2d, msg, zdp, part, shacc, idxst, mbuf, sem = refs
        core = lax.axis_index("c")
        sub = lax.axis_index("s")
        wid = sub * 2 + core
        c0 = wid * _CPW

        @pl.when(sub < 10)
        def _():
            rows = pl.ds(sub * 1000, 1000)
            pltpu.sync_copy(zdp.at[rows], shacc.at[rows])
            if with_counts:
                pltpu.sync_copy(z16.at[rows], shcnt.at[rows])
        if with_counts:
            pltpu.sync_copy(ones, onesbuf)
        pltpu.sync_copy(src2d.at[pl.ds(c0, _CPW)], idxst)
        plsc.subcore_barrier()

        def group(g, carry):
            cps = []
            for k in range(_GRP):
                c = c0 + g * _GRP + k
                cps.append(pltpu.async_copy(
                    msg.at[pl.ds(c * _CH, _CH)],
                    mbuf.at[pl.ds(k * _CH, _CH)], sem))
            for cp in cps:
                cp.wait()
            for k in range(_GRP):
                c = c0 + g * _GRP + k
                j = g * _GRP + k
                pltpu.sync_copy(mbuf.at[pl.ds(k * _CH, _CH)],
                                shacc.at[idxst.at[j]], add=True)
                if with_counts:
                    @pl.when(c < _NCHUNK)
                    def _(j=j):
                        pltpu.sync_copy(onesbuf, shcnt.at[idxst.at[j]], add=True)
            return carry

        lax.fori_loop(0, _NGRP, group, 0)
        plsc.subcore_barrier()

        @pl.when(sub < 10)
        def _():
            rows = pl.ds(sub * 1000, 1000)
            pltpu.sync_copy(shacc.at[rows], part.at[core, rows])
            if with_counts:
                pltpu.sync_copy(shcnt.at[rows], cpart.at[core, rows])

    return scatter


_SH_COEF = [
    1.0,
    float(np.sqrt(3.0)), float(np.sqrt(3.0)), float(np.sqrt(3.0)),
    float(np.sqrt(15.0)), float(np.sqrt(15.0)), float(np.sqrt(5.0) / 2.0),
    float(np.sqrt(15.0)), float(np.sqrt(15.0) / 2.0),
]
_G_STOP = 5.0
_G_COEF = float(-0.5 / ((_G_STOP / (_ES - 1)) ** 2))


def _pad_row_mask(msg):
    grow = (pl.program_id(0) * _BE
            + lax.broadcasted_iota(jnp.int32, msg.shape, 0))
    return jnp.where(grow < _E, msg, 0.0)


def _edge_mlp(elen, etype, hs16, hd16, fcW1, fcb1, fcW2, fcb2):
    pre = (jnp.dot(elen, fcW1[0:16], preferred_element_type=jnp.float32)
           + jnp.dot(etype, fcW1[16:32], preferred_element_type=jnp.float32)
           + jnp.dot(hs16, fcW1[32:48], preferred_element_type=jnp.float32)
           + jnp.dot(hd16, fcW1[48:64], preferred_element_type=jnp.float32)
           + fcb1)
    hid = jnp.maximum(pre, 0.0)
    return jnp.dot(hid, fcW2, preferred_element_type=jnp.float32) + fcb2


def _tp_msg(x, sh, wtp, dpout):
    acc = jnp.dot(x * sh[:, 0:1], wtp[0], preferred_element_type=jnp.float32)
    for k in range(1, _SH):
        acc = acc + jnp.dot(x * sh[:, k:k + 1], wtp[k],
                            preferred_element_type=jnp.float32)
    return acc


def _edge0_body(csrc_r, cdst_r, etype_r, hs_r, x_r,
                eW1_r, eb1_r, eW2_r, eb2_r,
                fcW1_r, fcb1_r, fcW2_r, fcb2_r, wtp_r,
                sh_o, elen_o, msg_o):
    ev = cdst_r[...] - csrc_r[...]
    n2 = jnp.sum(ev * ev, axis=1, keepdims=True)
    nrm = jnp.sqrt(n2)
    u = ev / jnp.maximum(nrm, 1e-8)
    ux, uy, uz = u[:, 0:1], u[:, 1:2], u[:, 2:3]
    cols = [jnp.ones_like(ux), ux, uy, uz,
            ux * uy, uy * uz, 3.0 * uz * uz - 1.0, ux * uz, ux * ux - uy * uy]
    lane = lax.broadcasted_iota(jnp.int32, (_BE, _ES), 1)
    sh = jnp.zeros((_BE, _ES), jnp.float32)
    for k in range(_SH):
        sh = jnp.where(lane == k, _SH_COEF[k] * cols[k], sh)
    sh_o[...] = sh

    offs = lane.astype(jnp.float32) * (_G_STOP / (_ES - 1))
    gg = jnp.exp(_G_COEF * (nrm - offs) ** 2)
    h1 = jnp.maximum(jnp.dot(gg, eW1_r[...],
                             preferred_element_type=jnp.float32) + eb1_r[...], 0.0)
    elen = jnp.dot(h1, eW2_r[...], preferred_element_type=jnp.float32) + eb2_r[...]
    elen_o[...] = elen

    x = x_r[...]
    w = _edge_mlp(elen, etype_r[...], hs_r[...], x[:, :16],
                  fcW1_r[...], fcb1_r[...], fcW2_r[...], fcb2_r[...])
    msg_o[...] = _pad_row_mask(_tp_msg(x, sh, wtp_r[...], _DPOUT[0]) * w)


def _edgeL_body(l, sh_r, elen_r, etype_r, hs_r, x_r,
                fcW1_r, fcb1_r, fcW2_r, fcb2_r, wtp_r, msg_o):
    x = x_r[...]
    sh = sh_r[...]
    w = _edge_mlp(elen_r[...], etype_r[...], hs_r[...], x[:, :16],
                  fcW1_r[...], fcb1_r[...], fcW2_r[...], fcb2_r[...])
    msg_o[...] = _pad_row_mask(_tp_msg(x, sh, wtp_r[...], _DPOUT[l]) * w)


def _full_spec(shape):
    return pl.BlockSpec(shape, lambda i: (0,) * len(shape))


def _rows_spec(w):
    return pl.BlockSpec((_BE, w), lambda i: (i, 0))


def _make_edge0(dpout):
    grid = (_EPAD // _BE,)
    in_specs = [_rows_spec(_ES)] * 5 + [
        _full_spec((_ES, _ES)), _full_spec((1, _ES)),
        _full_spec((_ES, _ES)), _full_spec((1, _ES)),
        _full_spec((64, 64)), _full_spec((1, 64)),
        _full_spec((64, dpout)), _full_spec((1, dpout)),
        _full_spec((_SH, _DPIN[0], dpout)),
    ]
    out_specs = [_rows_spec(_ES), _rows_spec(_ES), _rows_spec(dpout)]
    out_shape = [jax.ShapeDtypeStruct((_EPAD, _ES), jnp.float32),
                 jax.ShapeDtypeStruct((_EPAD, _ES), jnp.float32),
                 jax.ShapeDtypeStruct((_EPAD, dpout), jnp.float32)]
    return pl.pallas_call(_edge0_body, grid=grid, in_specs=in_specs,
                          out_specs=out_specs, out_shape=out_shape)


def _make_edgeL(l):
    dpin, dpout = _DPIN[l], _DPOUT[l]
    grid = (_EPAD // _BE,)
    in_specs = [_rows_spec(_ES)] * 4 + [
        _rows_spec(dpin),
        _full_spec((64, 64)), _full_spec((1, 64)),
        _full_spec((64, dpout)), _full_spec((1, dpout)),
        _full_spec((_SH, dpin, dpout)),
    ]
    out_specs = [_rows_spec(dpout)]
    out_shape = [jax.ShapeDtypeStruct((_EPAD, dpout), jnp.float32)]
    return pl.pallas_call(functools.partial(_edgeL_body, l), grid=grid,
                          in_specs=in_specs, out_specs=out_specs,
                          out_shape=out_shape)


def _norm_update(part, cnt, gamma, beta, dout, dpout):
    agg = part[0] + part[1]
    out = agg / cnt
    mu = jnp.sum(out, axis=1, keepdims=True) * (1.0 / dout)
    lane = lax.broadcasted_iota(jnp.int32, out.shape, 1)
    dev = jnp.where(lane < dout, out - mu, 0.0)
    var = jnp.sum(dev * dev, axis=1, keepdims=True) * (1.0 / dout)
    return dev * lax.rsqrt(var + 1e-5) * gamma + beta


def _node0_body(part_r, cpart_r, h_r, gamma_r, beta_r,
                h_o, h16_o, cnt_o):
    cnt = jnp.maximum(cpart_r[0] + cpart_r[1], 1.0)
    cnt_o[...] = cnt
    upd = _norm_update(part_r[...], cnt[:, 0:1], gamma_r[...], beta_r[...],
                       _DOUT[0], _DPOUT[0])
    hold = h_r[...]
    pad = _DPOUT[0] - hold.shape[1]
    hnew = jnp.concatenate(
        [hold, jnp.zeros((hold.shape[0], pad), jnp.float32)], axis=1) + upd
    h_o[...] = hnew
    h16_o[...] = hnew[:, :16]


def _nodeL_body(l, part_r, cnt_r, h_r, gamma_r, beta_r, h_o, h16_o):
    upd = _norm_update(part_r[...], cnt_r[:, 0:1], gamma_r[...], beta_r[...],
                       _DOUT[l], _DPOUT[l])
    hold = h_r[...]
    pad = _DPOUT[l] - hold.shape[1]
    if pad:
        hold = jnp.concatenate(
            [hold, jnp.zeros((hold.shape[0], pad), jnp.float32)], axis=1)
    hnew = hold + upd
    h_o[...] = hnew
    h16_o[...] = hnew[:, :16]


def _node3_body(part_r, cnt_r, h_r, gamma_r, beta_r,
                oW1_r, ob1_r, oW2_r, ob2_r, out_o):
    upd = _norm_update(part_r[...], cnt_r[:, 0:1], gamma_r[...], beta_r[...],
                       _DOUT[3], _DPOUT[3])
    hnew = h_r[...] + upd
    emb = jnp.concatenate([hnew[:, :16], hnew[:, 40:56]], axis=1)
    e1 = jnp.maximum(jnp.dot(emb, oW1_r[...],
                             preferred_element_type=jnp.float32) + ob1_r[...], 0.0)
    out_o[...] = jnp.dot(e1, oW2_r[...],
                         preferred_element_type=jnp.float32) + ob2_r[...]


def _nrows_spec(w):
    return pl.BlockSpec((_BN, w), lambda i: (i, 0))


def _part_spec(w):
    return pl.BlockSpec((2, _BN, w), lambda i: (0, i, 0))


def _make_node0():
    dpo = _DPOUT[0]
    grid = (_N // _BN,)
    in_specs = [_part_spec(dpo), _part_spec(_ES), _nrows_spec(_DPIN[0]),
                _full_spec((1, dpo)), _full_spec((1, dpo))]
    out_specs = [_nrows_spec(dpo), _nrows_spec(_ES), _nrows_spec(_ES)]
    out_shape = [jax.ShapeDtypeStruct((_N, dpo), jnp.float32),
                 jax.ShapeDtypeStruct((_N, _ES), jnp.float32),
                 jax.ShapeDtypeStruct((_N, _ES), jnp.float32)]
    return pl.pallas_call(_node0_body, grid=grid, in_specs=in_specs,
                          out_specs=out_specs, out_shape=out_shape)


def _make_nodeL(l):
    dpo = _DPOUT[l]
    grid = (_N // _BN,)
    in_specs = [_part_spec(dpo), _nrows_spec(_ES), _nrows_spec(_DPIN[l]),
                _full_spec((1, dpo)), _full_spec((1, dpo))]
    out_specs = [_nrows_spec(dpo), _nrows_spec(_ES)]
    out_shape = [jax.ShapeDtypeStruct((_N, dpo), jnp.float32),
                 jax.ShapeDtypeStruct((_N, _ES), jnp.float32)]
    return pl.pallas_call(functools.partial(_nodeL_body, l), grid=grid,
                          in_specs=in_specs, out_specs=out_specs,
                          out_shape=out_shape)


def _make_node3():
    dpo = _DPOUT[3]
    grid = (_N // _BN,)
    in_specs = [_part_spec(dpo), _nrows_spec(_ES), _nrows_spec(_DPIN[3]),
                _full_spec((1, dpo)), _full_spec((1, dpo)),
                _full_spec((2 * _NS, 2 * _NS)), _full_spec((1, 2 * _NS)),
                _full_spec((2 * _NS, _NS)), _full_spec((1, _NS))]
    out_specs = [_nrows_spec(_NS)]
    out_shape = [jax.ShapeDtypeStruct((_N, _NS), jnp.float32)]
    return pl.pallas_call(_node3_body, grid=grid, in_specs=in_specs,
                          out_specs=out_specs, out_shape=out_shape)


def _pad_cols(a, w):
    return jnp.pad(a, ((0, 0), (0, w - a.shape[1])))


def kernel(node_attr, coords, batch_id, perturb_mask, edges, edge_type_attr, params):
    src = edges[0].astype(jnp.int32)
    dst = edges[1].astype(jnp.int32)
    zpad = jnp.zeros((_EPAD - _E,), jnp.int32)
    src2d = jnp.concatenate([src, zpad]).reshape(_NCHUNK_PAD, _CH)
    dst2d = jnp.concatenate([dst, zpad]).reshape(_NCHUNK_PAD, _CH)
    coords_p = _pad_cols(coords.astype(jnp.float32), _ES)
    nat = node_attr.astype(jnp.float32)
    etype_p = jnp.pad(edge_type_attr, ((0, _EPAD - _E), (0, 0)))

    lw = []
    for l in range(_NLAYER):
        lp = params['layers']['l%d' % l]
        din, dout = _DIN[l], _DOUT[l]
        dpin, dpo = _DPIN[l], _DPOUT[l]
        wtp = lp['Wtp'].reshape(din, _SH, dout).transpose(1, 0, 2)
        wtp = jnp.pad(wtp, ((0, 0), (0, dpin - din), (0, dpo - dout)))
        lw.append(dict(
            fcW1=lp['fcW1'],
            fcb1=lp['fcb1'][None, :],
            fcW2=_pad_cols(lp['fcW2'], dpo),
            fcb2=_pad_cols(lp['fcb2'][None, :], dpo),
            wtp=wtp,
            gamma=_pad_cols(lp['gamma'][None, :], dpo),
            beta=_pad_cols(lp['beta'][None, :], dpo),
        ))
    ee = params['edge_emb']
    oo = params['out_ffn']

    ones16 = jnp.ones((_CH, _ES), jnp.float32)
    z16 = jnp.zeros((_N, _ES), jnp.float32)

    cdst, csrc, x0, hs0 = _make_gather([_ES] * 4, [0, 1, 0, 1])(
        dst2d, src2d, coords_p, coords_p, nat, nat)
    sh, elen, msg0 = _make_edge0(_DPOUT[0])(
        csrc, cdst, etype_p, hs0, x0,
        ee['W1'], ee['b1'][None, :], ee['W2'], ee['b2'][None, :],
        lw[0]['fcW1'], lw[0]['fcb1'], lw[0]['fcW2'], lw[0]['fcb2'],
        lw[0]['wtp'])
    part0, cpart = _make_scatter(_DPOUT[0], True)(
        src2d, msg0, jnp.zeros((_N, _DPOUT[0]), jnp.float32), z16, ones16)
    h, h16, cnt = _make_node0()(part0, cpart, nat, lw[0]['gamma'], lw[0]['beta'])

    for l in (1, 2):
        x, hs = _make_gather([_DPIN[l], _ES], [0, 1])(dst2d, src2d, h, h16)
        (msg,) = _make_edgeL(l)(
            sh, elen, etype_p, hs, x,
            lw[l]['fcW1'], lw[l]['fcb1'], lw[l]['fcW2'], lw[l]['fcb2'],
            lw[l]['wtp'])
        (part,) = _make_scatter(_DPOUT[l], False)(
            src2d, msg, jnp.zeros((_N, _DPOUT[l]), jnp.float32))
        h, h16 = _make_nodeL(l)(part, cnt, h, lw[l]['gamma'], lw[l]['beta'])

    x, hs = _make_gather([_DPIN[3], _ES], [0, 1])(dst2d, src2d, h, h16)
    (msg,) = _make_edgeL(3)(
        sh, elen, etype_p, hs, x,
        lw[3]['fcW1'], lw[3]['fcb1'], lw[3]['fcW2'], lw[3]['fcb2'],
        lw[3]['wtp'])
    (part,) = _make_scatter(_DPOUT[3], False)(
        src2d, msg, jnp.zeros((_N, _DPOUT[3]), jnp.float32))
    (out,) = _make_node3()(part, cnt, h, lw[3]['gamma'], lw[3]['beta'],
                           oo['W1'], oo['b1'][None, :], oo['W2'], oo['b2'][None, :])
    return out
